# baseline (speedup 1.0000x reference)
import jax
import jax.numpy as jnp
from jax import lax
from jax.experimental import pallas as pl


_N_EMB = 1024


def _vq_body(xa_ref, wa_ref, w_ref, res_ref, idx_ref):
    xa = xa_ref[...]
    wa = wa_ref[...]
    M = xa.shape[0]
    dist = lax.dot_general(
        xa, wa, (((1,), (0,)), ((), ())),
        preferred_element_type=jnp.float32,
        precision=lax.Precision.HIGHEST,
    )
    m = jnp.min(dist, axis=1, keepdims=True)
    ids = lax.broadcasted_iota(jnp.int32, (M, _N_EMB), 1)
    idx = jnp.min(jnp.where(dist == m, ids, jnp.int32(_N_EMB)),
                  axis=1, keepdims=True)
    idx_ref[...] = idx
    onehot = (ids == idx).astype(jnp.float32)
    w = w_ref[...]
    B = res_ref.shape[0]
    P = M // B
    for b in range(B):
        res_ref[b] = lax.dot_general(
            w, onehot[b * P:(b + 1) * P], (((1,), (1,)), ((), ())),
            preferred_element_type=jnp.float32,
            precision=lax.Precision.HIGHEST,
        )


def kernel(x, weight):
    B, D, H, W = x.shape
    P = H * W
    M = B * P
    xt = x.reshape(B, D, P).transpose(0, 2, 1).reshape(M, D)
    xa = jnp.concatenate(
        [xt, jnp.ones((M, 1), jnp.float32), jnp.zeros((M, 7), jnp.float32)],
        axis=1)
    e2 = jnp.sum(weight * weight, axis=0, keepdims=True)
    wa = jnp.concatenate(
        [-2.0 * weight, e2, jnp.zeros((7, _N_EMB), jnp.float32)], axis=0)
    res, idx = pl.pallas_call(
        _vq_body,
        out_shape=[
            jax.ShapeDtypeStruct((B, D, P), jnp.float32),
            jax.ShapeDtypeStruct((M, 1), jnp.int32),
        ],
    )(xa, wa, weight)
    return res.reshape(B, D, H, W), idx.reshape(B, H, W)
